# Initial kernel scaffold; baseline (speedup 1.0000x reference)
import jax
import jax.numpy as jnp
from jax.experimental import pallas as pl


def kernel(x, edge_index, batch, W_enc, b_enc, p_global, P, W1, b1, w2, b2, W_g, b_g, Wp, bp):
    raise NotImplementedError("write your pallas kernel here")


# speedup vs baseline: 6.0022x; 6.0022x over previous
import functools

import jax
import jax.numpy as jnp
from jax import lax
from jax.experimental import pallas as pl
from jax.experimental.pallas import tpu as pltpu
from jax.experimental.pallas import tpu_sc as plsc

W_MIX = 1.0
G = 64
CHUNK = 128


def _k1_body(x_ref, we_ref, be_ref, wg_ref, bg_ref, h0_ref, hg_ref):
    xb = x_ref[...]
    h0_ref[...] = jnp.dot(xb, we_ref[...], preferred_element_type=jnp.float32) + be_ref[...]
    hg_ref[...] = jnp.dot(xb, wg_ref[...], preferred_element_type=jnp.float32) + bg_ref[...]


def _k1(x_pad, W_enc, b_enc2, W_g, b_g2, blk):
    npad, d = x_pad.shape
    grid = (npad // blk,)
    return pl.pallas_call(
        _k1_body,
        grid=grid,
        in_specs=[
            pl.BlockSpec((blk, d), lambda i: (i, 0)),
            pl.BlockSpec((d, d), lambda i: (0, 0)),
            pl.BlockSpec((1, d), lambda i: (0, 0)),
            pl.BlockSpec((d, d), lambda i: (0, 0)),
            pl.BlockSpec((1, d), lambda i: (0, 0)),
        ],
        out_specs=[
            pl.BlockSpec((blk, d), lambda i: (i, 0)),
            pl.BlockSpec((blk, d), lambda i: (i, 0)),
        ],
        out_shape=[
            jax.ShapeDtypeStruct((npad, d), jnp.float32),
            jax.ShapeDtypeStruct((npad, d), jnp.float32),
        ],
    )(x_pad, W_enc, b_enc2, W_g, b_g2)


def _sc_segsum(table, srcl, dstl, npad, nbranches, d, with_deg=False):
    cw = srcl.shape[2]
    half = npad // 2
    acc_rows = half + CHUNK
    rpt = acc_rows // 16
    wpt = half // 16
    zrows = 8
    nout = nbranches + (1 if with_deg else 0)
    mesh = plsc.VectorSubcoreMesh(core_axis_name="c", subcore_axis_name="s")

    @functools.partial(
        pl.kernel,
        mesh=mesh,
        out_type=jax.ShapeDtypeStruct((2, nout, half, d), jnp.float32),
        scratch_types=[
            pltpu.VMEM((cw, CHUNK), jnp.int32),
            pltpu.VMEM((cw, CHUNK), jnp.int32),
            pltpu.VMEM((CHUNK, d), jnp.float32),
            pltpu.VMEM((CHUNK, d), jnp.float32),
            pltpu.VMEM((zrows, d), jnp.float32),
            pltpu.VMEM_SHARED((acc_rows, d), jnp.float32),
            pltpu.SemaphoreType.DMA,
            pltpu.SemaphoreType.DMA,
        ],
    )
    def k(table_hbm, src_hbm, dst_hbm, out_hbm, src_v, dst_v, bufa, bufb, zb,
          accum, sema, semb):
        c = lax.axis_index("c")
        s = lax.axis_index("s")

        zero16 = jnp.zeros((16,), jnp.float32)
        for r in range(zrows):
            for kk in range(d // 16):
                zb[r, pl.ds(16 * kk, 16)] = zero16

        pltpu.sync_copy(dst_hbm.at[s], dst_v)
        base = jnp.full((16,), c * half, jnp.int32)
        jbase = jnp.full((16,), half, jnp.int32)
        m127 = jnp.full((16,), CHUNK - 1, jnp.int32)
        hlim = jnp.full((16,), half, jnp.int32)

        def rbody(m, carry):
            j = m // (CHUNK // 16)
            kk = m % (CHUNK // 16)
            dd = dst_v[j, pl.ds(16 * kk, 16)]
            tt = dd - base
            inr = (tt >= 0) & (tt < hlim)
            dst_v[j, pl.ds(16 * kk, 16)] = jnp.where(
                inr, tt, jbase + (dd & m127))
            return carry
        lax.fori_loop(0, cw * (CHUNK // 16), rbody, 0)

        def zero_accum_slab():
            def zbody(kk, carry):
                pltpu.sync_copy(zb, accum.at[pl.ds(s * rpt + kk * zrows, zrows)])
                return carry
            lax.fori_loop(0, rpt // zrows, zbody, 0)
            plsc.subcore_barrier()

        def writeback(i):
            plsc.subcore_barrier()
            pltpu.sync_copy(
                accum.at[pl.ds(s * wpt, wpt)],
                out_hbm.at[c, i, pl.ds(s * wpt, wpt)],
            )

        def start_gather(j, buf, sem):
            return pltpu.async_copy(table_hbm.at[src_v.at[j]], buf, sem)

        def wait_gather(buf, sem):
            pltpu.make_async_copy(table_hbm.at[src_v.at[0]], buf, sem).wait()

        def scat_add(buf, j):
            pltpu.sync_copy(buf, accum.at[dst_v.at[j]], add=True)

        for i in range(nbranches):
            zero_accum_slab()
            pltpu.sync_copy(src_hbm.at[i, s], src_v)

            start_gather(0, bufa, sema)
            start_gather(1, bufb, semb)

            def ebody(jj, carry):
                j0 = 2 * jj
                wait_gather(bufa, sema)
                scat_add(bufa, j0)
                start_gather(j0 + 2, bufa, sema)
                wait_gather(bufb, semb)
                scat_add(bufb, j0 + 1)
                start_gather(j0 + 3, bufb, semb)
                return carry
            lax.fori_loop(0, cw // 2 - 1, ebody, 0)

            wait_gather(bufa, sema)
            scat_add(bufa, cw - 2)
            wait_gather(bufb, semb)
            scat_add(bufb, cw - 1)

            writeback(i)

        if with_deg:
            one16 = jnp.ones((16,), jnp.float32)
            for r in range(CHUNK):
                for kk in range(d // 16):
                    bufa[r, pl.ds(16 * kk, 16)] = one16
            zero_accum_slab()

            def dbody(jj, carry):
                for u in range(8):
                    pltpu.async_copy(bufa, accum.at[dst_v.at[8 * jj + u]],
                                     sema, add=True)
                for u in range(8):
                    pltpu.make_async_copy(bufa, accum.at[dst_v.at[0]],
                                          sema).wait()
                return carry
            lax.fori_loop(0, cw // 8, dbody, 0)
            writeback(nbranches)

    return k(table, srcl, dstl)


def _k3_body(part_ref, hg_ref, w1_ref, b1_ref, p_ref, pg_ref, wt2_ref, b2_ref,
             att_ref, u_ref):
    d = hg_ref.shape[1]
    agg0 = part_ref[0, 0]
    degf = part_ref[0, 1]
    deg = jnp.sum(degf, axis=1, keepdims=True) * (1.0 / d)
    emb = jax.nn.relu(agg0 / jnp.maximum(deg, 1.0))

    a = 1.0 / (1.0 + W_MIX)
    bw = W_MIX / (1.0 + W_MIX)
    cvec = a * p_ref[...] + bw * pg_ref[...]
    dshift = jnp.dot(cvec, w1_ref[...], preferred_element_type=jnp.float32)

    z = jnp.dot(emb, w1_ref[...], preferred_element_type=jnp.float32) + b1_ref[...]
    r = jax.nn.relu(z[None, :, :] + dshift[:, None, :])
    logit = jnp.sum(r * wt2_ref[...][None, :, :], axis=2) + b2_ref[0, 0]
    att = jax.nn.sigmoid(logit)
    att_ref[...] = att
    u_ref[...] = hg_ref[...][None, :, :] * att[:, :, None]


def _k3(part1, hg, W1, b12, P, pg2, wt2, b22, blk):
    npad, d = hg.shape
    t = P.shape[0]
    hb = part1.shape[2] // blk
    grid = (npad // blk,)
    return pl.pallas_call(
        _k3_body,
        grid=grid,
        in_specs=[
            pl.BlockSpec((1, 2, blk, d), lambda i: (i // hb, 0, i % hb, 0)),
            pl.BlockSpec((blk, d), lambda i: (i, 0)),
            pl.BlockSpec((d, d), lambda i: (0, 0)),
            pl.BlockSpec((1, d), lambda i: (0, 0)),
            pl.BlockSpec((t, d), lambda i: (0, 0)),
            pl.BlockSpec((1, d), lambda i: (0, 0)),
            pl.BlockSpec((1, d), lambda i: (0, 0)),
            pl.BlockSpec((1, 1), lambda i: (0, 0)),
        ],
        out_specs=[
            pl.BlockSpec((t, blk), lambda i: (0, i)),
            pl.BlockSpec((t, blk, d), lambda i: (0, i, 0)),
        ],
        out_shape=[
            jax.ShapeDtypeStruct((t, npad), jnp.float32),
            jax.ShapeDtypeStruct((t, npad, d), jnp.float32),
        ],
    )(part1, hg, W1, b12, P, pg2, wt2, b22)


def _k5_body(part_ref, att_ref, batch_ref, wp_ref, bp_ref, out_ref, acc_ref):
    i = pl.program_id(0)
    n = pl.num_programs(0)
    t, blk = att_ref.shape
    s = part_ref[0]
    v = jnp.sum(jax.nn.relu(s) * wp_ref[...][:, None, :], axis=2)
    q = att_ref[...] * v
    ones_row = jnp.ones((1, blk), jnp.float32)
    zeros_rows = jnp.zeros((8 - t - 1, blk), jnp.float32)
    q8 = jnp.concatenate([q, ones_row, zeros_rows], axis=0)

    bb = batch_ref[...].reshape(1, blk)
    gi = lax.broadcasted_iota(jnp.int32, (G, blk), 0)
    oh = (bb == gi).astype(jnp.float32)
    partial = lax.dot_general(oh, q8, (((1,), (1,)), ((), ())),
                              preferred_element_type=jnp.float32)

    @pl.when(i == 0)
    def _():
        acc_ref[...] = partial

    @pl.when(i > 0)
    def _():
        acc_ref[...] += partial

    @pl.when(i == n - 1)
    def _():
        acc = acc_ref[...]
        pooled = acc[:, :t] / jnp.maximum(acc[:, t:t + 1], 1.0)
        out_ref[...] = pooled + bp_ref[...]


def _k5(part2, att, batch3d, wp2, bp2, npad, blk):
    t, half, d = part2.shape[1:]
    hb = half // blk
    grid = (npad // blk,)
    return pl.pallas_call(
        _k5_body,
        grid=grid,
        in_specs=[
            pl.BlockSpec((1, t, blk, d), lambda i: (i // hb, 0, i % hb, 0)),
            pl.BlockSpec((t, blk), lambda i: (0, i)),
            pl.BlockSpec((1, 1, blk), lambda i: (i, 0, 0)),
            pl.BlockSpec((t, d), lambda i: (0, 0)),
            pl.BlockSpec((1, t), lambda i: (0, 0)),
        ],
        out_specs=pl.BlockSpec((G, t), lambda i: (0, 0)),
        out_shape=jax.ShapeDtypeStruct((G, t), jnp.float32),
        scratch_shapes=[pltpu.VMEM((G, 8), jnp.float32)],
    )(part2, att, batch3d, wp2, bp2)


def kernel(x, edge_index, batch, W_enc, b_enc, p_global, P, W1, b1, w2, b2,
           W_g, b_g, Wp, bp):
    n, d = x.shape
    e = edge_index.shape[1]
    t = P.shape[0]

    ns = 16
    blk = 1024
    npad = ((n + blk - 1) // blk) * blk
    eps = ((e + ns * 8 * CHUNK - 1) // (ns * 8 * CHUNK)) * 8 * CHUNK
    cw = eps // CHUNK
    ep = ns * eps

    src = edge_index[0]
    dst = edge_index[1]
    pad_s = eps - e // ns
    src_p = jnp.concatenate(
        [src.reshape(ns, e // ns),
         jnp.zeros((ns, pad_s), jnp.int32)], axis=1)
    junk = (1 << 20) + (jnp.arange(pad_s, dtype=jnp.int32) % CHUNK)
    dst_p = jnp.concatenate(
        [dst.reshape(ns, e // ns),
         jnp.broadcast_to(junk, (ns, pad_s))], axis=1)
    dstl = dst_p.reshape(ns, cw, CHUNK)
    srcl1 = src_p.reshape(1, ns, cw, CHUNK)
    offs = (jnp.arange(t, dtype=jnp.int32) * npad)[:, None, None]
    srcl2 = src_p[None] + offs
    srcl2 = srcl2.reshape(t, ns, cw, CHUNK)

    x_pad = jnp.zeros((npad, d), jnp.float32).at[:n].set(x)
    batch_pad = jnp.full((npad,), G, jnp.int32).at[:n].set(batch)
    batch3d = batch_pad.reshape(npad // blk, 1, blk)

    h0, hg = _k1(x_pad, W_enc, b_enc.reshape(1, d), W_g, b_g.reshape(1, d), blk)

    part1 = _sc_segsum(h0, srcl1, dstl, npad, 1, d, with_deg=True)

    att, u = _k3(part1, hg, W1, b1.reshape(1, d), P,
                 p_global.reshape(1, d), w2.reshape(1, d), b2.reshape(1, 1), blk)

    part2 = _sc_segsum(u.reshape(t * npad, d), srcl2, dstl, npad, t, d)

    out = _k5(part2, att, batch3d, Wp.reshape(t, d), bp.reshape(1, t), npad, blk)
    return out
